# baseline (speedup 1.0000x reference)
import functools

import jax
import jax.numpy as jnp
from jax import lax
from jax.experimental import pallas as pl
from jax.experimental.pallas import tpu as pltpu
from jax.experimental.pallas import tpu_sc as plsc

D_MODEL = 1024
VOCAB_ROWS = 50
VOCAB_PAD = 64
B_TOTAL = 4 * 8192
_LANES = 16

N_SC = 16384
N_TC = B_TOTAL - N_SC
_TC_BLOCK = 2048

_INFO = plsc.get_sparse_core_info()
_NC = _INFO.num_cores
_NS = _INFO.num_subcores
_NW = _NC * _NS
_B_PER_W = N_SC // _NW
_NGROUP = _B_PER_W // _LANES


def _make_sc_lookup():
    mesh = plsc.VectorSubcoreMesh(core_axis_name="c", subcore_axis_name="s")

    @functools.partial(
        pl.kernel,
        mesh=mesh,
        out_type=jax.ShapeDtypeStruct((N_SC, D_MODEL), jnp.float32),
        scratch_types=[
            pltpu.VMEM((_B_PER_W,), jnp.int32),
            pltpu.VMEM((VOCAB_ROWS, D_MODEL), jnp.float32),
            pltpu.SemaphoreType.DMA,
        ],
    )
    def sc_lookup(table_hbm, idx_hbm, out_hbm, idx_v, table_v, sem):
        wid = lax.axis_index("s") * _NC + lax.axis_index("c")
        base = wid * _B_PER_W
        pltpu.sync_copy(table_hbm, table_v)
        pltpu.sync_copy(idx_hbm.at[pl.ds(base, _B_PER_W)], idx_v)

        @pl.loop(0, _NGROUP)
        def _(g):
            r0 = g * _LANES
            idx16 = idx_v[pl.ds(r0, _LANES)]
            for l in range(_LANES):
                pltpu.async_copy(
                    table_v.at[idx16[l]], out_hbm.at[base + r0 + l], sem
                )

        @pl.loop(0, _NGROUP)
        def _(g):
            pltpu.make_async_copy(
                table_v.at[pl.ds(0, _LANES)],
                out_hbm.at[pl.ds(base, _LANES)],
                sem,
            ).wait()

    return sc_lookup


_sc_lookup = _make_sc_lookup()


def _tc_body(idx_ref, table_ref, out_ref):
    idx = idx_ref[0, 0, :]
    onehot = (
        idx[:, None]
        == jax.lax.broadcasted_iota(jnp.int32, (_TC_BLOCK, VOCAB_PAD), 1)
    ).astype(jnp.float32)
    out_ref[...] = jnp.dot(
        onehot, table_ref[...], preferred_element_type=jnp.float32
    )


_tc_lookup = pl.pallas_call(
    _tc_body,
    grid=(N_TC // _TC_BLOCK,),
    in_specs=[
        pl.BlockSpec((1, 1, _TC_BLOCK), lambda i: (i, 0, 0)),
        pl.BlockSpec((VOCAB_PAD, D_MODEL), lambda i: (0, 0)),
    ],
    out_specs=pl.BlockSpec((_TC_BLOCK, D_MODEL), lambda i: (i, 0)),
    out_shape=jax.ShapeDtypeStruct((N_TC, D_MODEL), jnp.float32),
)


@jax.jit
def kernel(x, table):
    flat_idx = x.reshape(B_TOTAL).astype(jnp.int32)
    sc_out = _sc_lookup(table, flat_idx[:N_SC])
    table_pad = jnp.pad(table, ((0, VOCAB_PAD - VOCAB_ROWS), (0, 0)))
    tc_idx = flat_idx[N_SC:].reshape(N_TC // _TC_BLOCK, 1, _TC_BLOCK)
    tc_out = _tc_lookup(tc_idx, table_pad)
    out = jnp.concatenate([sc_out, tc_out], axis=0)
    return out.reshape(x.shape[0], x.shape[1], D_MODEL)


# speedup vs baseline: 2.2363x; 2.2363x over previous
import functools

import jax
import jax.numpy as jnp
from jax import lax
from jax.experimental import pallas as pl
from jax.experimental.pallas import tpu as pltpu
from jax.experimental.pallas import tpu_sc as plsc

D_MODEL = 1024
VOCAB_ROWS = 50
B_TOTAL = 4 * 8192
_LANES = 16

_INFO = plsc.get_sparse_core_info()
_NC = _INFO.num_cores
_NS = _INFO.num_subcores
_NW = _NC * _NS
_B_PER_W = B_TOTAL // _NW
_NGROUP = _B_PER_W // _LANES


def _make_sc_lookup():
    mesh = plsc.VectorSubcoreMesh(core_axis_name="c", subcore_axis_name="s")

    @functools.partial(
        pl.kernel,
        mesh=mesh,
        out_type=jax.ShapeDtypeStruct((B_TOTAL, D_MODEL), jnp.float32),
        scratch_types=[
            pltpu.VMEM((_B_PER_W,), jnp.int32),
            pltpu.VMEM((VOCAB_ROWS, D_MODEL), jnp.float32),
            pltpu.SemaphoreType.DMA,
        ],
    )
    def sc_lookup(table_hbm, idx_hbm, out_hbm, idx_v, table_v, sem):
        wid = lax.axis_index("s") * _NC + lax.axis_index("c")
        base = wid * _B_PER_W
        pltpu.sync_copy(table_hbm, table_v)
        pltpu.sync_copy(idx_hbm.at[pl.ds(base, _B_PER_W)], idx_v)

        @pl.loop(0, _NGROUP)
        def _(g):
            r0 = g * _LANES
            idx16 = idx_v[pl.ds(r0, _LANES)]
            for l in range(_LANES):
                pltpu.async_copy(
                    table_v.at[idx16[l]], out_hbm.at[base + r0 + l], sem
                )

        @pl.loop(0, _NGROUP)
        def _(g):
            pltpu.make_async_copy(
                table_v.at[pl.ds(0, _LANES)],
                out_hbm.at[pl.ds(base, _LANES)],
                sem,
            ).wait()

    return sc_lookup


_sc_lookup = _make_sc_lookup()


@jax.jit
def kernel(x, table):
    flat_idx = x.reshape(B_TOTAL).astype(jnp.int32)
    out = _sc_lookup(table, flat_idx)
    return out.reshape(x.shape[0], x.shape[1], D_MODEL)
